# baseline (speedup 1.0000x reference)
import functools

import jax
import jax.numpy as jnp
from jax import lax
from jax.experimental import pallas as pl
from jax.experimental.pallas import tpu as pltpu
from jax.experimental.pallas import tpu_sc as plsc

VOCAB = 100000
EMBED = 32
CTX = 20
BATCH = 1024

NUM_CORES = 2
NUM_SUBCORES = 16
NW = NUM_CORES * NUM_SUBCORES
BPW = BATCH // NW
IDX_PER_W = BPW * CTX
IDX_CHUNK = 128
N_CHUNKS = IDX_PER_W // IDX_CHUNK
LANES = 16

VB = 4096
N_VBLK = (VOCAB + VB - 1) // VB


def _sc_pool_body(x_hbm, table_hbm, out_hbm, idx_v, rows_v, acc_v, sem):
    wid = lax.axis_index("s") * NUM_CORES + lax.axis_index("c")
    base = wid * BPW
    pltpu.sync_copy(x_hbm.at[pl.ds(base * CTX, IDX_PER_W)], idx_v)
    copies = []
    for j in range(N_CHUNKS):
        copies.append(pltpu.async_copy(
            table_hbm.at[idx_v.at[pl.ds(j * IDX_CHUNK, IDX_CHUNK)]],
            rows_v.at[pl.ds(j * IDX_CHUNK, IDX_CHUNK)],
            sem,
        ))
    for c in copies:
        c.wait()

    def body(b, carry):
        r0 = b * CTX
        acc0 = rows_v[r0, pl.ds(0, LANES)]
        acc1 = rows_v[r0, pl.ds(LANES, LANES)]
        for c in range(1, CTX):
            acc0 = acc0 + rows_v[r0 + c, pl.ds(0, LANES)]
            acc1 = acc1 + rows_v[r0 + c, pl.ds(LANES, LANES)]
        acc_v[b, pl.ds(0, LANES)] = acc0
        acc_v[b, pl.ds(LANES, LANES)] = acc1
        return carry

    lax.fori_loop(0, BPW, body, 0)
    pltpu.sync_copy(acc_v, out_hbm.at[pl.ds(base, BPW)])


def _sc_pool(x_flat, table_lin):
    mesh = plsc.VectorSubcoreMesh(core_axis_name="c", subcore_axis_name="s")
    fn = functools.partial(
        pl.kernel,
        mesh=mesh,
        out_type=jax.ShapeDtypeStruct((BATCH, EMBED), jnp.float32),
        scratch_types=[
            pltpu.VMEM((IDX_PER_W,), jnp.int32),
            pltpu.VMEM((IDX_PER_W, EMBED), jnp.float32),
            pltpu.VMEM((BPW, EMBED), jnp.float32),
            pltpu.SemaphoreType.DMA,
        ],
        compiler_params=pltpu.CompilerParams(use_tc_tiling_on_sc=False),
    )(_sc_pool_body)
    return fn(x_flat, table_lin)


def _tc_proj_body(pooled_ref, wt_ref, b_ref, out_ref):
    mm = lax.dot_general(
        wt_ref[...], pooled_ref[...],
        (((0,), (0,)), ((), ())),
        preferred_element_type=jnp.float32,
    )
    ones = jnp.ones((1, BATCH), jnp.float32)
    bias = lax.dot_general(
        b_ref[...], ones,
        (((0,), (0,)), ((), ())),
        preferred_element_type=jnp.float32,
    )
    out_ref[...] = mm + bias


def _tc_project(pooled_t, WT, b2):
    return pl.pallas_call(
        _tc_proj_body,
        grid=(N_VBLK,),
        in_specs=[
            pl.BlockSpec((EMBED, BATCH), lambda i: (0, 0)),
            pl.BlockSpec((EMBED, VB), lambda i: (0, i)),
            pl.BlockSpec((1, VB), lambda i: (0, i)),
        ],
        out_specs=pl.BlockSpec((VB, BATCH), lambda i: (i, 0)),
        out_shape=jax.ShapeDtypeStruct((VOCAB, BATCH), jnp.float32),
    )(pooled_t, WT, b2)


def kernel(x, emb_table, W, b):
    x_flat = x.reshape(-1).astype(jnp.int32)
    table_lin = lax.optimization_barrier(
        emb_table.reshape(-1)
    ).reshape(VOCAB, EMBED)
    pooled = _sc_pool(x_flat, table_lin)
    logits_t = _tc_project(pooled.T, W.T, b.reshape(1, VOCAB))
    return logits_t.T


# speedup vs baseline: 1.0383x; 1.0383x over previous
import functools

import jax
import jax.numpy as jnp
from jax import lax
from jax.experimental import pallas as pl
from jax.experimental.pallas import tpu as pltpu
from jax.experimental.pallas import tpu_sc as plsc

VOCAB = 100000
EMBED = 32
CTX = 20
BATCH = 1024

NUM_CORES = 2
NUM_SUBCORES = 16
NW = NUM_CORES * NUM_SUBCORES
BPW = BATCH // NW
IDX_PER_W = BPW * CTX
IDX_CHUNK = 128
N_CHUNKS = IDX_PER_W // IDX_CHUNK
LANES = 16

VB = 2048
N_VBLK = (VOCAB + VB - 1) // VB

DCHUNK = 4096
N_DBLK = (VOCAB + DCHUNK - 1) // DCHUNK


def _sc_pool_body(x_hbm, table_hbm, out_hbm, idx_v, rows_v, acc_v, sem):
    wid = lax.axis_index("s") * NUM_CORES + lax.axis_index("c")
    base = wid * BPW
    pltpu.sync_copy(x_hbm.at[pl.ds(base * CTX, IDX_PER_W)], idx_v)
    copies = []
    for j in range(N_CHUNKS):
        copies.append(pltpu.async_copy(
            table_hbm.at[idx_v.at[pl.ds(j * IDX_CHUNK, IDX_CHUNK)]],
            rows_v.at[pl.ds(j * IDX_CHUNK, IDX_CHUNK)],
            sem,
        ))
    for c in copies:
        c.wait()

    def body(b, carry):
        r0 = b * CTX
        acc0 = rows_v[r0, pl.ds(0, LANES)]
        acc1 = rows_v[r0, pl.ds(LANES, LANES)]
        for c in range(1, CTX):
            acc0 = acc0 + rows_v[r0 + c, pl.ds(0, LANES)]
            acc1 = acc1 + rows_v[r0 + c, pl.ds(LANES, LANES)]
        acc_v[b, pl.ds(0, LANES)] = acc0
        acc_v[b, pl.ds(LANES, LANES)] = acc1
        return carry

    lax.fori_loop(0, BPW, body, 0)
    pltpu.sync_copy(acc_v, out_hbm.at[pl.ds(base, BPW)])


def _sc_pool(x_flat, table_lin):
    mesh = plsc.VectorSubcoreMesh(core_axis_name="c", subcore_axis_name="s")
    fn = functools.partial(
        pl.kernel,
        mesh=mesh,
        out_type=jax.ShapeDtypeStruct((BATCH, EMBED), jnp.float32),
        scratch_types=[
            pltpu.VMEM((IDX_PER_W,), jnp.int32),
            pltpu.VMEM((IDX_PER_W, EMBED), jnp.float32),
            pltpu.VMEM((BPW, EMBED), jnp.float32),
            pltpu.SemaphoreType.DMA,
        ],
        compiler_params=pltpu.CompilerParams(use_tc_tiling_on_sc=False),
    )(_sc_pool_body)
    return fn(x_flat, table_lin)


def _tc_depad_body(embt_ref, out_ref):
    t = jnp.transpose(embt_ref[...], (1, 0))
    u = t.reshape(DCHUNK // 4, 4, EMBED)
    out_ref[...] = jnp.concatenate(
        [u[:, 0, :], u[:, 1, :], u[:, 2, :], u[:, 3, :]], axis=1)


def _tc_depad(emb_t):
    return pl.pallas_call(
        _tc_depad_body,
        grid=(N_DBLK,),
        in_specs=[pl.BlockSpec((EMBED, DCHUNK), lambda i: (0, i))],
        out_specs=pl.BlockSpec((DCHUNK // 4, 128), lambda i: (i, 0)),
        out_shape=jax.ShapeDtypeStruct(
            (VOCAB * EMBED // 128, 128), jnp.float32),
    )(emb_t)


def _tc_proj_body(pooled_ref, wt_ref, b_ref, out_ref):
    mm = lax.dot_general(
        wt_ref[...], pooled_ref[...],
        (((0,), (0,)), ((), ())),
        preferred_element_type=jnp.float32,
    )
    ones = jnp.ones((1, BATCH), jnp.float32)
    bias = lax.dot_general(
        b_ref[...], ones,
        (((0,), (0,)), ((), ())),
        preferred_element_type=jnp.float32,
    )
    out_ref[...] = mm + bias


def _tc_project(pooled_t, WT, b2):
    return pl.pallas_call(
        _tc_proj_body,
        grid=(N_VBLK,),
        in_specs=[
            pl.BlockSpec((EMBED, BATCH), lambda i: (0, 0)),
            pl.BlockSpec((EMBED, VB), lambda i: (0, i)),
            pl.BlockSpec((1, VB), lambda i: (0, i)),
        ],
        out_specs=pl.BlockSpec((VB, BATCH), lambda i: (i, 0)),
        out_shape=jax.ShapeDtypeStruct((VOCAB, BATCH), jnp.float32),
    )(pooled_t, WT, b2)


def kernel(x, emb_table, W, b):
    x_flat = x.reshape(-1).astype(jnp.int32)
    table_lin = _tc_depad(emb_table.T).reshape(-1).reshape(VOCAB, EMBED)
    pooled = _sc_pool(x_flat, table_lin)
    logits_t = _tc_project(pooled.T, W.T, b.reshape(1, VOCAB))
    return logits_t.T


# speedup vs baseline: 1.0446x; 1.0060x over previous
import functools

import jax
import jax.numpy as jnp
from jax import lax
from jax.experimental import pallas as pl
from jax.experimental.pallas import tpu as pltpu
from jax.experimental.pallas import tpu_sc as plsc

VOCAB = 100000
EMBED = 32
CTX = 20
BATCH = 1024

NUM_CORES = 2
NUM_SUBCORES = 16
NW = NUM_CORES * NUM_SUBCORES
BPW = BATCH // NW
IDX_PER_W = BPW * CTX
IDX_CHUNK = 128
N_CHUNKS = IDX_PER_W // IDX_CHUNK
LANES = 16

VB = 2048
N_VBLK = (VOCAB + VB - 1) // VB

DCHUNK = 8192
N_DBLK = (VOCAB + DCHUNK - 1) // DCHUNK


def _sc_pool_body(x_hbm, table_hbm, out_hbm, idx_v, rows_v, acc_v, sem):
    wid = lax.axis_index("s") * NUM_CORES + lax.axis_index("c")
    base = wid * BPW
    pltpu.sync_copy(x_hbm.at[pl.ds(base * CTX, IDX_PER_W)], idx_v)
    copies = []
    for j in range(N_CHUNKS):
        copies.append(pltpu.async_copy(
            table_hbm.at[idx_v.at[pl.ds(j * IDX_CHUNK, IDX_CHUNK)]],
            rows_v.at[pl.ds(j * IDX_CHUNK, IDX_CHUNK)],
            sem,
        ))
    for c in copies:
        c.wait()

    def body(b, carry):
        r0 = b * CTX
        acc0 = rows_v[r0, pl.ds(0, LANES)]
        acc1 = rows_v[r0, pl.ds(LANES, LANES)]
        for c in range(1, CTX):
            acc0 = acc0 + rows_v[r0 + c, pl.ds(0, LANES)]
            acc1 = acc1 + rows_v[r0 + c, pl.ds(LANES, LANES)]
        acc_v[b, pl.ds(0, LANES)] = acc0
        acc_v[b, pl.ds(LANES, LANES)] = acc1
        return carry

    lax.fori_loop(0, BPW, body, 0)
    pltpu.sync_copy(acc_v, out_hbm.at[pl.ds(base, BPW)])


def _sc_pool(x_flat, table_lin):
    mesh = plsc.VectorSubcoreMesh(core_axis_name="c", subcore_axis_name="s")
    fn = functools.partial(
        pl.kernel,
        mesh=mesh,
        out_type=jax.ShapeDtypeStruct((BATCH, EMBED), jnp.float32),
        scratch_types=[
            pltpu.VMEM((IDX_PER_W,), jnp.int32),
            pltpu.VMEM((IDX_PER_W, EMBED), jnp.float32),
            pltpu.VMEM((BPW, EMBED), jnp.float32),
            pltpu.SemaphoreType.DMA,
        ],
        compiler_params=pltpu.CompilerParams(use_tc_tiling_on_sc=False),
    )(_sc_pool_body)
    return fn(x_flat, table_lin)


def _tc_depad_body(embt_ref, out_ref):
    t = jnp.transpose(embt_ref[...], (1, 0))
    u = t.reshape(DCHUNK // 4, 4, EMBED)
    out_ref[...] = jnp.concatenate(
        [u[:, 0, :], u[:, 1, :], u[:, 2, :], u[:, 3, :]], axis=1)


def _tc_depad(emb_t):
    return pl.pallas_call(
        _tc_depad_body,
        grid=(N_DBLK,),
        in_specs=[pl.BlockSpec((EMBED, DCHUNK), lambda i: (0, i))],
        out_specs=pl.BlockSpec((DCHUNK // 4, 128), lambda i: (i, 0)),
        out_shape=jax.ShapeDtypeStruct(
            (VOCAB * EMBED // 128, 128), jnp.float32),
    )(emb_t)


def _tc_proj_body(pooled_ref, wt_ref, b_ref, out_ref):
    mm = lax.dot_general(
        wt_ref[...], pooled_ref[...],
        (((0,), (0,)), ((), ())),
        preferred_element_type=jnp.float32,
    )
    ones = jnp.ones((1, BATCH), jnp.float32)
    bias = lax.dot_general(
        b_ref[...], ones,
        (((0,), (0,)), ((), ())),
        preferred_element_type=jnp.float32,
    )
    out_ref[...] = mm + bias


def _tc_project(pooled_t, WT, b2):
    return pl.pallas_call(
        _tc_proj_body,
        grid=(N_VBLK,),
        in_specs=[
            pl.BlockSpec((EMBED, BATCH), lambda i: (0, 0)),
            pl.BlockSpec((EMBED, VB), lambda i: (0, i)),
            pl.BlockSpec((1, VB), lambda i: (0, i)),
        ],
        out_specs=pl.BlockSpec((VB, BATCH), lambda i: (i, 0)),
        out_shape=jax.ShapeDtypeStruct((VOCAB, BATCH), jnp.float32),
    )(pooled_t, WT, b2)


def kernel(x, emb_table, W, b):
    x_flat = x.reshape(-1).astype(jnp.int32)
    table_lin = _tc_depad(emb_table.T).reshape(-1).reshape(VOCAB, EMBED)
    pooled = _sc_pool(x_flat, table_lin)
    logits_t = _tc_project(pooled.T, W.T, b.reshape(1, VOCAB))
    return logits_t.T
